# Initial kernel scaffold; baseline (speedup 1.0000x reference)
import jax
import jax.numpy as jnp
from jax.experimental import pallas as pl


def kernel(nodes, neigh_idx, u2e, W1, b1, W2, b2, W3, b3):
    raise NotImplementedError("write your pallas kernel here")


# speedup vs baseline: 3.2176x; 3.2176x over previous
import functools

import jax
import jax.numpy as jnp
from jax import lax
from jax.experimental import pallas as pl
from jax.experimental.pallas import tpu as pltpu
from jax.experimental.pallas import tpu_sc as plsc

D = 128
N_NODES = 10000
DEG = 32

NC = 2
NS = 16
NW = NC * NS

CH = 120
B_EDGES = N_NODES * DEG
_B_RAW = B_EDGES + N_NODES
_GRAN = NW * CH
B_TOTAL = ((_B_RAW + _GRAN - 1) // _GRAN) * _GRAN
BPW = B_TOTAL // NW
NCHUNK = BPW // CH
NPAIR = NCHUNK // 2

BN = 80
GRID = N_NODES // BN
UOFF = B_EDGES // BN


def _sc_gather_body(table_hbm, idx_hbm, out_hbm, idx_v, buf0, buf1, sem0, sem1):
    c = lax.axis_index("c")
    s = lax.axis_index("s")
    wid = s * NC + c
    base = wid * BPW
    pltpu.sync_copy(idx_hbm.at[pl.ds(base, BPW)], idx_v)

    def gather(j, buf, sem):
        return pltpu.make_async_copy(
            table_hbm.at[idx_v.at[pl.ds(j * CH, CH)]], buf, sem)

    def put(j, buf):
        pltpu.sync_copy(buf, out_hbm.at[pl.ds(base + j * CH, CH)])

    gather(0, buf0, sem0).start()

    def pair_body(p, carry):
        a = 2 * p
        gather(a, buf0, sem0).wait()
        gather(a + 1, buf1, sem1).start()
        put(a, buf0)
        gather(a + 1, buf1, sem1).wait()

        @pl.when(p + 1 < NPAIR)
        def _():
            gather(a + 2, buf0, sem0).start()

        put(a + 1, buf1)
        return carry

    lax.fori_loop(0, NPAIR, pair_body, 0)


def _make_sc_gather():
    return functools.partial(
        pl.kernel,
        mesh=plsc.VectorSubcoreMesh(core_axis_name="c", subcore_axis_name="s"),
        out_type=jax.ShapeDtypeStruct((B_TOTAL, D), jnp.float32),
        scratch_types=[
            pltpu.VMEM((BPW,), jnp.int32),
            pltpu.VMEM((CH, D), jnp.float32),
            pltpu.VMEM((CH, D), jnp.float32),
            pltpu.SemaphoreType.DMA,
            pltpu.SemaphoreType.DMA,
        ],
    )(_sc_gather_body)


def _tc_mlp_body(e_ref, u_ref, w1a_ref, w1b_ref, b1_ref, w2_ref, b2_ref,
                 w3t_ref, out_ref):
    e = e_ref[...]
    u = u_ref[...]
    h1 = jnp.dot(e, w1a_ref[...], preferred_element_type=jnp.float32)
    hu = jnp.dot(u, w1b_ref[...], preferred_element_type=jnp.float32)
    hu = hu + b1_ref[...]
    h1 = h1.reshape(BN, DEG, D) + hu[:, None, :]
    h1 = jnp.maximum(h1, 0.0).reshape(BN * DEG, D)
    h2 = jnp.dot(h1, w2_ref[...], preferred_element_type=jnp.float32)
    h2 = jnp.maximum(h2 + b2_ref[...], 0.0)
    logits = jnp.sum(h2 * w3t_ref[...], axis=1, keepdims=True)
    l3 = logits.reshape(BN, DEG, 1)
    m = jnp.max(l3, axis=1, keepdims=True)
    p = jnp.exp(l3 - m)
    att = p / jnp.sum(p, axis=1, keepdims=True)
    out_ref[...] = jnp.sum(e.reshape(BN, DEG, D) * att, axis=1)


_tc_mlp = pl.pallas_call(
    _tc_mlp_body,
    grid=(GRID,),
    in_specs=[
        pl.BlockSpec((BN * DEG, D), lambda i: (i, 0)),
        pl.BlockSpec((BN, D), lambda i: (UOFF + i, 0)),
        pl.BlockSpec((D, D), lambda i: (0, 0)),
        pl.BlockSpec((D, D), lambda i: (0, 0)),
        pl.BlockSpec((1, D), lambda i: (0, 0)),
        pl.BlockSpec((D, D), lambda i: (0, 0)),
        pl.BlockSpec((1, D), lambda i: (0, 0)),
        pl.BlockSpec((1, D), lambda i: (0, 0)),
    ],
    out_specs=pl.BlockSpec((BN, D), lambda i: (i, 0)),
    out_shape=jax.ShapeDtypeStruct((N_NODES, D), jnp.float32),
    compiler_params=pltpu.CompilerParams(
        dimension_semantics=("arbitrary",)),
)


def kernel(nodes, neigh_idx, u2e, W1, b1, W2, b2, W3, b3):
    pad = B_TOTAL - B_EDGES - N_NODES
    idx_all = jnp.concatenate([
        neigh_idx.reshape(-1),
        nodes,
        jnp.zeros((pad,), jnp.int32),
    ])
    gathered = _make_sc_gather()(u2e, idx_all)
    out = _tc_mlp(
        gathered,
        gathered,
        W1[:D],
        W1[D:],
        b1.reshape(1, D),
        W2,
        b2.reshape(1, D),
        W3.reshape(1, D),
    )
    return out
